# baseline (speedup 1.0000x reference)
import jax
import jax.numpy as jnp
import numpy as np
from jax import lax
from jax.experimental import pallas as pl
from jax.experimental.pallas import tpu as pltpu
from jax.experimental.pallas import tpu_sc as plsc

_NC = 2
_NS = 16
_L = 16
_NW = _NC * _NS
_W = 512
_NROWS = (16 * 512 * 512) // _W
_ROWS_PER_W = _NROWS // _NW
_BLK = 32
_NCHUNK = _ROWS_PER_W // _BLK
_SLICES = _W // _L

_MU = (0.93, 0.9, 0.88, 0.85, 0.87, 0.85, 0.92, 0.91, 0.96, 0.95, 0.85,
       0.98, 0.97, 0.25, 0.3, 0.28, 0.27, 0.25, 0.28)
_SD = (0.03, 0.05, 0.06, 0.08, 0.07, 0.05, 0.04, 0.04, 0.02, 0.03, 0.1,
       0.01, 0.01, 0.1, 0.12, 0.11, 0.1, 0.1, 0.09)
_MARGIN = 1.5
_BIG = 1.0e9

_TBL = np.array(
    [(m - _MARGIN * s if s < 0.12 else -_BIG) for m, s in zip(_MU, _SD)]
    + [-_BIG] * (32 - len(_MU))
    + [(m + _MARGIN * s if s < 0.12 else _BIG) for m, s in zip(_MU, _SD)]
    + [_BIG] * (32 - len(_SD)),
    dtype=np.float32)


def _sc_body(e_hbm, seg_hbm, tbl_hbm, out_hbm,
             lo_v, hi_v, e_v0, e_v1, s_v0, s_v1, o_v,
             sem_e0, sem_e1, sem_s0, sem_s1):
    cid = lax.axis_index("c")
    sid = lax.axis_index("s")
    wid = sid * _NC + cid
    base = wid * _ROWS_PER_W

    pltpu.sync_copy(tbl_hbm.at[pl.ds(0, 32)], lo_v)
    pltpu.sync_copy(tbl_hbm.at[pl.ds(32, 32)], hi_v)

    sem_e = (sem_e0, sem_e1)
    sem_s = (sem_s0, sem_s1)
    e_v = (e_v0, e_v1)
    s_v = (s_v0, s_v1)

    def fire(ci, b):
        r0 = base + jnp.minimum(ci, _NCHUNK - 1) * _BLK
        pltpu.async_copy(e_hbm.at[pl.ds(r0, _BLK)], e_v[b], sem_e[b])
        pltpu.async_copy(seg_hbm.at[pl.ds(r0, _BLK)], s_v[b], sem_s[b])

    def wait(b):
        pltpu.make_async_copy(e_hbm.at[pl.ds(0, _BLK)], e_v[b],
                              sem_e[b]).wait()
        pltpu.make_async_copy(seg_hbm.at[pl.ds(0, _BLK)], s_v[b],
                              sem_s[b]).wait()

    def compute(b, acc):
        eb = e_v[b]
        sb = s_v[b]

        def body(r, c):
            accs = list(c)
            for k in range(_SLICES):
                j = k % 2
                a_s, a_c = accs[j]
                idx = sb[r, pl.ds(k * _L, _L)]
                e = eb[r, pl.ds(k * _L, _L)]
                lo = plsc.load_gather(lo_v, [idx])
                hi = plsc.load_gather(hi_v, [idx])
                p = jnp.maximum(jnp.maximum(lo - e, e - hi), 0.0)
                m = jnp.where(hi < 2.0, 1.0, 0.0).astype(jnp.float32)
                accs[j] = (a_s + p, a_c + m)
            return tuple(accs)

        return plsc.parallel_loop(0, _BLK, 1, unroll=2, carry=acc)(body)

    zero = jnp.zeros((_L,), jnp.float32)
    acc0 = ((zero, zero), (zero, zero))
    fire(0, 0)

    def pair(pi, acc):
        ci = 2 * pi
        fire(ci + 1, 1)
        wait(0)
        acc = compute(0, acc)
        fire(ci + 2, 0)
        wait(1)
        return compute(1, acc)

    acc = lax.fori_loop(0, _NCHUNK // 2, pair, acc0)
    wait(0)

    (s0, c0), (s1, c1) = acc
    o_v[pl.ds(0, _L)] = s0 + s1
    pltpu.sync_copy(o_v, out_hbm.at[pl.ds(wid * _L, _L)])
    o_v[pl.ds(0, _L)] = c0 + c1
    pltpu.sync_copy(o_v, out_hbm.at[pl.ds(_NW * _L + wid * _L, _L)])


_sc_call = pl.kernel(
    _sc_body,
    out_type=jax.ShapeDtypeStruct((2 * _NW * _L,), jnp.float32),
    mesh=plsc.VectorSubcoreMesh(core_axis_name="c", subcore_axis_name="s"),
    compiler_params=pltpu.CompilerParams(needs_layout_passes=False),
    scratch_types=[
        pltpu.VMEM((32,), jnp.float32),
        pltpu.VMEM((32,), jnp.float32),
        pltpu.VMEM((_BLK, _W), jnp.float32),
        pltpu.VMEM((_BLK, _W), jnp.float32),
        pltpu.VMEM((_BLK, _W), jnp.int32),
        pltpu.VMEM((_BLK, _W), jnp.int32),
        pltpu.VMEM((_L,), jnp.float32),
        pltpu.SemaphoreType.DMA,
        pltpu.SemaphoreType.DMA,
        pltpu.SemaphoreType.DMA,
        pltpu.SemaphoreType.DMA,
    ],
)


def kernel(e_pred, seg_mask):
    e = e_pred.reshape(_NROWS, _W)
    seg = seg_mask.reshape(_NROWS, _W).astype(jnp.int32)
    part = _sc_call(e, seg, jnp.asarray(_TBL))
    psum = jnp.sum(part[: _NW * _L])
    total = jnp.sum(part[_NW * _L:])
    return jnp.where(total < 1.0, jnp.float32(0.0),
                     psum / jnp.maximum(total, 1.0))


# speedup vs baseline: 2.7970x; 2.7970x over previous
import jax
import jax.numpy as jnp
import numpy as np
from jax import lax
from jax.experimental import pallas as pl
from jax.experimental.pallas import tpu as pltpu
from jax.experimental.pallas import tpu_sc as plsc

_NC = 2
_NS = 16
_L = 16
_NW = _NC * _NS
_N = 16 * 512 * 512
_W = 512
_NROWS = _N // _W
_BLK = 8
_ROWS_PER_W = _NROWS // _NW
_NCHUNK = _ROWS_PER_W // _BLK

_MU = (0.93, 0.9, 0.88, 0.85, 0.87, 0.85, 0.92, 0.91, 0.96, 0.95, 0.85,
       0.98, 0.97, 0.25, 0.3, 0.28, 0.27, 0.25, 0.28)
_SD = (0.03, 0.05, 0.06, 0.08, 0.07, 0.05, 0.04, 0.04, 0.02, 0.03, 0.1,
       0.01, 0.01, 0.1, 0.12, 0.11, 0.1, 0.1, 0.09)
_MARGIN = 1.5
_BIG = 1.0e9

_TBL = np.array(
    [(m - _MARGIN * s if s < 0.12 else -_BIG) for m, s in zip(_MU, _SD)]
    + [-_BIG] * (32 - len(_MU))
    + [(m + _MARGIN * s if s < 0.12 else _BIG) for m, s in zip(_MU, _SD)]
    + [_BIG] * (32 - len(_SD)),
    dtype=np.float32)


def _sc_body(e_hbm, seg_hbm, tbl_hbm, out_hbm,
             lo_v, hi_v, e_v0, e_v1, s_v0, s_v1, o_v,
             sem_e0, sem_e1, sem_s0, sem_s1):
    cid = lax.axis_index("c")
    sid = lax.axis_index("s")
    wid = sid * _NC + cid
    base = wid * _ROWS_PER_W

    e_hbm = e_hbm.reshape(_NROWS, _W)
    seg_hbm = seg_hbm.reshape(_NROWS, _W)

    pltpu.sync_copy(tbl_hbm.at[pl.ds(0, 32)], lo_v)
    pltpu.sync_copy(tbl_hbm.at[pl.ds(32, 32)], hi_v)

    sem_e = (sem_e0, sem_e1)
    sem_s = (sem_s0, sem_s1)
    e_v = (e_v0, e_v1)
    s_v = (s_v0, s_v1)

    def fire(ci, b):
        r0 = base + jnp.minimum(ci, _NCHUNK - 1) * _BLK
        pltpu.async_copy(e_hbm.at[pl.ds(r0, _BLK)], e_v[b], sem_e[b])
        pltpu.async_copy(seg_hbm.at[pl.ds(r0, _BLK)], s_v[b], sem_s[b])

    def wait(b):
        pltpu.make_async_copy(e_hbm.at[pl.ds(0, _BLK)], e_v[b],
                              sem_e[b]).wait()
        pltpu.make_async_copy(seg_hbm.at[pl.ds(0, _BLK)], s_v[b],
                              sem_s[b]).wait()

    def compute(b, acc):
        eb = e_v[b]
        sb = s_v[b]
        for row in range(_BLK):
            def body(i, c, row=row):
                out = []
                for j, (a_s, a_c) in enumerate(c):
                    idx = sb[row, pl.ds(i + j * _L, _L)]
                    e = eb[row, pl.ds(i + j * _L, _L)]
                    lo = plsc.load_gather(lo_v, [idx])
                    hi = plsc.load_gather(hi_v, [idx])
                    p = jnp.maximum(jnp.maximum(lo - e, e - hi), 0.0)
                    m = jnp.where(hi < 2.0, 1.0, 0.0).astype(jnp.float32)
                    out.append((a_s + p, a_c + m))
                return tuple(out)

            acc = plsc.parallel_loop(0, _W, 2 * _L, unroll=4,
                                     carry=acc)(body)
        return acc

    zero = jnp.zeros((_L,), jnp.float32)
    acc0 = ((zero, zero), (zero, zero))
    fire(0, 0)

    def pair(pi, acc):
        ci = 2 * pi
        fire(ci + 1, 1)
        wait(0)
        acc = compute(0, acc)
        fire(ci + 2, 0)
        wait(1)
        return compute(1, acc)

    acc = lax.fori_loop(0, _NCHUNK // 2, pair, acc0)
    wait(0)

    (s0, c0), (s1, c1) = acc
    o_v[pl.ds(0, _L)] = s0 + s1
    pltpu.sync_copy(o_v, out_hbm.at[pl.ds(wid * _L, _L)])
    o_v[pl.ds(0, _L)] = c0 + c1
    pltpu.sync_copy(o_v, out_hbm.at[pl.ds(_NW * _L + wid * _L, _L)])


_sc_call = pl.kernel(
    _sc_body,
    out_type=jax.ShapeDtypeStruct((2 * _NW * _L,), jnp.float32),
    mesh=plsc.VectorSubcoreMesh(core_axis_name="c", subcore_axis_name="s"),
    compiler_params=pltpu.CompilerParams(needs_layout_passes=False),
    scratch_types=[
        pltpu.VMEM((32,), jnp.float32),
        pltpu.VMEM((32,), jnp.float32),
        pltpu.VMEM((_BLK, _W), jnp.float32),
        pltpu.VMEM((_BLK, _W), jnp.float32),
        pltpu.VMEM((_BLK, _W), jnp.int32),
        pltpu.VMEM((_BLK, _W), jnp.int32),
        pltpu.VMEM((_L,), jnp.float32),
        pltpu.SemaphoreType.DMA,
        pltpu.SemaphoreType.DMA,
        pltpu.SemaphoreType.DMA,
        pltpu.SemaphoreType.DMA,
    ],
)


def kernel(e_pred, seg_mask):
    part = _sc_call(e_pred, seg_mask.astype(jnp.int32), jnp.asarray(_TBL))
    psum = jnp.sum(part[: _NW * _L])
    total = jnp.sum(part[_NW * _L:])
    return jnp.where(total < 1.0, jnp.float32(0.0),
                     psum / jnp.maximum(total, 1.0))
